# Initial kernel scaffold; baseline (speedup 1.0000x reference)
import jax
import jax.numpy as jnp
from jax.experimental import pallas as pl


def kernel(images, flows):
    raise NotImplementedError("write your pallas kernel here")


# speedup vs baseline: 1.1921x; 1.1921x over previous
import functools

import jax
import jax.numpy as jnp
from jax import lax
from jax.experimental import pallas as pl
from jax.experimental.pallas import tpu as pltpu
from jax.experimental.pallas import tpu_sc as plsc

B, C, H, W = 64, 3, 512, 512
HW = H * W
NW = 32
P = 2048
CHUNKS_PER_IMG = HW // P
TOTAL_CHUNKS = B * CHUNKS_PER_IMG
CHUNKS_PER_WORKER = TOTAL_CHUNKS // NW
G = P // 16


def _sc_warp(images_hbm, flows_hbm, out_hbm, fl_v, idx_v, tap_v, w_v, o_v, sem):
    wid = lax.axis_index("s") * 2 + lax.axis_index("c")

    def chunk_body(k, carry):
        chunk_id = wid * CHUNKS_PER_WORKER + k
        b = chunk_id >> 7
        off = (chunk_id & 127) << 11
        base = b * (3 * HW)

        fy_off = pl.multiple_of(2 * b * HW + off, 8)
        fx_off = pl.multiple_of((2 * b + 1) * HW + off, 8)
        pltpu.sync_copy(flows_hbm.at[pl.ds(fy_off, P)], fl_v.at[0])
        pltpu.sync_copy(flows_hbm.at[pl.ds(fx_off, P)], fl_v.at[1])

        def compute_body(g, carry):
            s = g * 16
            lane = lax.iota(jnp.int32, 16)
            hw = off + s + lane
            yi = hw >> 9
            xi = hw & 511
            fy = fl_v[0, pl.ds(s, 16)]
            fx = fl_v[1, pl.ds(s, 16)]
            sy = jnp.clip(yi.astype(jnp.float32) + fy, 0.0, float(H) - 1.0)
            sx = jnp.clip(xi.astype(jnp.float32) + fx, 0.0, float(W) - 1.0)
            y0 = jnp.minimum(sy.astype(jnp.int32), H - 2)
            x0 = jnp.minimum(sx.astype(jnp.int32), W - 2)
            ay = sy - y0.astype(jnp.float32)
            ax = sx - x0.astype(jnp.float32)
            by = 1.0 - ay
            bx = 1.0 - ax
            w_v[0, pl.ds(s, 16)] = bx * by
            w_v[1, pl.ds(s, 16)] = bx * ay
            w_v[2, pl.ds(s, 16)] = ax * by
            w_v[3, pl.ds(s, 16)] = ax * ay
            ia = base + y0 * W + x0
            idx_v[0, pl.ds(s, 16)] = ia
            idx_v[1, pl.ds(s, 16)] = ia + W
            idx_v[2, pl.ds(s, 16)] = ia + 1
            idx_v[3, pl.ds(s, 16)] = ia + (W + 1)
            ib = ia + HW
            idx_v[4, pl.ds(s, 16)] = ib
            idx_v[5, pl.ds(s, 16)] = ib + W
            idx_v[6, pl.ds(s, 16)] = ib + 1
            idx_v[7, pl.ds(s, 16)] = ib + (W + 1)
            ic = ib + HW
            idx_v[8, pl.ds(s, 16)] = ic
            idx_v[9, pl.ds(s, 16)] = ic + W
            idx_v[10, pl.ds(s, 16)] = ic + 1
            idx_v[11, pl.ds(s, 16)] = ic + (W + 1)
            return carry

        lax.fori_loop(0, G, compute_body, 0)

        copies = [
            pltpu.async_copy(images_hbm.at[idx_v.at[t]], tap_v.at[t], sem)
            for t in range(12)
        ]
        for cp in copies:
            cp.wait()

        def combine_body(g, carry):
            s = g * 16
            wa = w_v[0, pl.ds(s, 16)]
            wb = w_v[1, pl.ds(s, 16)]
            wc = w_v[2, pl.ds(s, 16)]
            wd = w_v[3, pl.ds(s, 16)]
            for c in range(3):
                acc = wa * tap_v[4 * c + 0, pl.ds(s, 16)]
                acc = acc + wb * tap_v[4 * c + 1, pl.ds(s, 16)]
                acc = acc + wc * tap_v[4 * c + 2, pl.ds(s, 16)]
                acc = acc + wd * tap_v[4 * c + 3, pl.ds(s, 16)]
                o_v[c, pl.ds(s, 16)] = acc
            return carry

        lax.fori_loop(0, G, combine_body, 0)

        for c in range(3):
            o_off = pl.multiple_of((3 * b + c) * HW + off, 8)
            pltpu.sync_copy(o_v.at[c], out_hbm.at[pl.ds(o_off, P)])
        return carry

    lax.fori_loop(0, CHUNKS_PER_WORKER, chunk_body, 0)


@jax.jit
def kernel(images, flows):
    images_flat = images.reshape(B * C * HW)
    flows_flat = flows.reshape(B * 2 * HW)
    mesh = plsc.VectorSubcoreMesh(core_axis_name="c", subcore_axis_name="s")
    run = functools.partial(
        pl.kernel,
        mesh=mesh,
        out_type=jax.ShapeDtypeStruct((B * C * HW,), jnp.float32),
        scratch_types=[
            pltpu.VMEM((2, P), jnp.float32),
            pltpu.VMEM((12, P), jnp.int32),
            pltpu.VMEM((12, P), jnp.float32),
            pltpu.VMEM((4, P), jnp.float32),
            pltpu.VMEM((3, P), jnp.float32),
            pltpu.SemaphoreType.DMA,
        ],
        compiler_params=pltpu.CompilerParams(use_tc_tiling_on_sc=False),
    )(_sc_warp)
    out = run(images_flat, flows_flat)
    return out.reshape(B, C, H, W)


# speedup vs baseline: 3.1388x; 2.6329x over previous
import functools

import jax
import jax.numpy as jnp
from jax import lax
from jax.experimental import pallas as pl
from jax.experimental.pallas import tpu as pltpu
from jax.experimental.pallas import tpu_sc as plsc

B, C, H, W = 64, 3, 512, 512
HW = H * W
NW = 32
IMGS_PER_WORKER = B // NW
P = 1024
ROWS_PER_CHUNK = P // W
NCH = HW // P
G = P // 16
RB = P + W + 16
D = 16


def _sc_warp(images_hbm, flows_hbm, out_hbm, tab_hbm,
             row_v, stage_v, fl_v, w_v, idx_v, o_v,
             sf0, sf1, sg0, sg1, so0, so1):
    SF = (sf0, sf1)
    SG = (sg0, sg1)
    SO = (so0, so1)
    wid = lax.axis_index("s") * 2 + lax.axis_index("c")

    def img_body(i, carry):
        b = wid * IMGS_PER_WORKER + i

        def in_copies(k, p):
            r0 = k * ROWS_PER_CHUNK
            cps = []
            for c in range(3):
                src = pl.multiple_of((b * 3 + c) * HW + r0 * W, 8)
                cps.append(pltpu.make_async_copy(
                    images_hbm.at[pl.ds(src, P)],
                    row_v.at[p, c, pl.ds(0, P)], SF[p]))
                nxt = jnp.minimum(r0 + ROWS_PER_CHUNK, H - 1) * W
                srcn = pl.multiple_of((b * 3 + c) * HW + nxt, 8)
                cps.append(pltpu.make_async_copy(
                    images_hbm.at[pl.ds(srcn, W)],
                    row_v.at[p, c, pl.ds(P, W)], SF[p]))
            return cps

        def tab_out_copy(k, p):
            dst = pl.multiple_of(b * HW + k * P, 8)
            return pltpu.make_async_copy(
                stage_v.at[p], tab_hbm.at[pl.ds(dst, P), :], SO[p])

        def interleave(p):
            def build_group(g, carry):
                s = g * 16
                lane = lax.iota(jnp.int32, 16)
                rowidx = s + lane
                for c in range(3):
                    for t, off in enumerate((0, 1, W, W + 1)):
                        v = row_v[p, c, pl.ds(s + off, 16)]
                        plsc.store_scatter(
                            stage_v.at[p],
                            [rowidx, jnp.full((16,), 4 * c + t, jnp.int32)], v)
                return carry
            lax.fori_loop(0, G, build_group, 0)

        for cp in in_copies(0, 0):
            cp.start()
        for cp in in_copies(1, 1):
            cp.start()

        def build_pair(j, carry):
            for p in (0, 1):
                k = 2 * j + p
                for cp in in_copies(k, p):
                    cp.wait()

                @pl.when(j >= 1)
                def _():
                    tab_out_copy(k - 2, p).wait()

                interleave(p)
                tab_out_copy(k, p).start()

                @pl.when(j < NCH // 2 - 1)
                def _():
                    for cp in in_copies(k + 2, p):
                        cp.start()
            return carry

        lax.fori_loop(0, NCH // 2, build_pair, 0)
        tab_out_copy(NCH - 2, 0).wait()
        tab_out_copy(NCH - 1, 1).wait()

        def flow_copies(k, p):
            off = k * P
            fy = pl.multiple_of(2 * b * HW + off, 8)
            fx = pl.multiple_of((2 * b + 1) * HW + off, 8)
            return [
                pltpu.make_async_copy(flows_hbm.at[pl.ds(fy, P)],
                                      fl_v.at[p, 0], SF[p]),
                pltpu.make_async_copy(flows_hbm.at[pl.ds(fx, P)],
                                      fl_v.at[p, 1], SF[p]),
            ]

        def gather_copy(p):
            return pltpu.make_async_copy(
                tab_hbm.at[idx_v.at[p]], stage_v.at[p], SG[p])

        def out_copies(k, p):
            off = k * P
            cps = []
            for c in range(3):
                dst = pl.multiple_of((3 * b + c) * HW + off, 8)
                cps.append(pltpu.make_async_copy(
                    o_v.at[p, c], out_hbm.at[pl.ds(dst, P)], SO[p]))
            return cps

        def compute(k, p):
            off = k * P

            def compute_group(g, carry):
                s = g * 16
                lane = lax.iota(jnp.int32, 16)
                hw = off + s + lane
                yi = hw >> 9
                xi = hw & 511
                fy = fl_v[p, 0, pl.ds(s, 16)]
                fx = fl_v[p, 1, pl.ds(s, 16)]
                sy = jnp.clip(yi.astype(jnp.float32) + fy, 0.0, float(H) - 1.0)
                sx = jnp.clip(xi.astype(jnp.float32) + fx, 0.0, float(W) - 1.0)
                y0 = jnp.minimum(sy.astype(jnp.int32), H - 2)
                x0 = jnp.minimum(sx.astype(jnp.int32), W - 2)
                ay = sy - y0.astype(jnp.float32)
                ax = sx - x0.astype(jnp.float32)
                by = 1.0 - ay
                bx = 1.0 - ax
                w_v[p, 0, pl.ds(s, 16)] = bx * by
                w_v[p, 1, pl.ds(s, 16)] = ax * by
                w_v[p, 2, pl.ds(s, 16)] = bx * ay
                w_v[p, 3, pl.ds(s, 16)] = ax * ay
                idx_v[p, pl.ds(s, 16)] = b * HW + y0 * W + x0
                return carry

            lax.fori_loop(0, G, compute_group, 0)

        def combine(p):
            def combine_group(g, carry):
                s = g * 16
                lane = lax.iota(jnp.int32, 16)
                rowidx = s + lane
                wa = w_v[p, 0, pl.ds(s, 16)]
                wc = w_v[p, 1, pl.ds(s, 16)]
                wb = w_v[p, 2, pl.ds(s, 16)]
                wd = w_v[p, 3, pl.ds(s, 16)]
                for c in range(3):
                    def tap(t):
                        col = jnp.full((16,), 4 * c + t, jnp.int32)
                        return plsc.load_gather(stage_v.at[p], [rowidx, col])
                    acc = wa * tap(0)
                    acc = acc + wc * tap(1)
                    acc = acc + wb * tap(2)
                    acc = acc + wd * tap(3)
                    o_v[p, c, pl.ds(s, 16)] = acc
                return carry

            lax.fori_loop(0, G, combine_group, 0)

        for cp in flow_copies(0, 0):
            cp.start()

        def sample_pair(j, carry):
            for p in (0, 1):
                k = 2 * j + p
                for cp in flow_copies(k, p):
                    cp.wait()
                compute(k, p)
                gather_copy(p).start()

                if p == 0:
                    for cp in flow_copies(k + 1, 1):
                        cp.start()
                else:
                    @pl.when(j < NCH // 2 - 1)
                    def _():
                        for cp in flow_copies(k + 1, 0):
                            cp.start()

                q = 1 - p
                if p == 0:
                    @pl.when(j >= 1)
                    def _():
                        gather_copy(q).wait()

                        @pl.when(j >= 2)
                        def _():
                            for cp in out_copies(k - 3, q):
                                cp.wait()

                        combine(q)
                        for cp in out_copies(k - 1, q):
                            cp.start()
                else:
                    gather_copy(q).wait()

                    @pl.when(j >= 1)
                    def _():
                        for cp in out_copies(k - 3, q):
                            cp.wait()

                    combine(q)
                    for cp in out_copies(k - 1, q):
                        cp.start()
            return carry

        lax.fori_loop(0, NCH // 2, sample_pair, 0)

        gather_copy(1).wait()
        for cp in out_copies(NCH - 3, 1):
            cp.wait()
        combine(1)
        for cp in out_copies(NCH - 1, 1):
            cp.start()
        for cp in out_copies(NCH - 2, 0):
            cp.wait()
        for cp in out_copies(NCH - 1, 1):
            cp.wait()
        return carry

    lax.fori_loop(0, IMGS_PER_WORKER, img_body, 0)


@jax.jit
def kernel(images, flows):
    images_flat = images.reshape(B * C * HW)
    flows_flat = flows.reshape(B * 2 * HW)
    mesh = plsc.VectorSubcoreMesh(core_axis_name="c", subcore_axis_name="s")
    run = functools.partial(
        pl.kernel,
        mesh=mesh,
        out_type=(
            jax.ShapeDtypeStruct((B * C * HW,), jnp.float32),
            jax.ShapeDtypeStruct((B * HW, D), jnp.float32),
        ),
        scratch_types=[
            pltpu.VMEM((2, 3, RB), jnp.float32),
            pltpu.VMEM((2, P, D), jnp.float32),
            pltpu.VMEM((2, 2, P), jnp.float32),
            pltpu.VMEM((2, 4, P), jnp.float32),
            pltpu.VMEM((2, P), jnp.int32),
            pltpu.VMEM((2, 3, P), jnp.float32),
            pltpu.SemaphoreType.DMA,
            pltpu.SemaphoreType.DMA,
            pltpu.SemaphoreType.DMA,
            pltpu.SemaphoreType.DMA,
            pltpu.SemaphoreType.DMA,
            pltpu.SemaphoreType.DMA,
        ],
        compiler_params=pltpu.CompilerParams(
            use_tc_tiling_on_sc=False, needs_layout_passes=False),
    )(_sc_warp)
    out, _tab = run(images_flat, flows_flat)
    return out.reshape(B, C, H, W)


# speedup vs baseline: 3.9012x; 1.2429x over previous
import functools

import jax
import jax.numpy as jnp
from jax import lax
from jax.experimental import pallas as pl
from jax.experimental.pallas import tpu as pltpu
from jax.experimental.pallas import tpu_sc as plsc

B, C, H, W = 64, 3, 512, 512
HW = H * W
NW = 32
IMGS_PER_WORKER = B // NW
P = 2048
ROWS_PER_CHUNK = P // W
NCH = HW // P
G = P // 16
RB = P + W + 16
D = 16


def _sc_warp(images_hbm, flows_hbm, out_hbm, tab_hbm,
             row_v, stage_v, fl_v, w_v, idx_v, o_v,
             sf0, sf1, sg0, sg1, so0, so1):
    SF = (sf0, sf1)
    SG = (sg0, sg1)
    SO = (so0, so1)
    wid = lax.axis_index("s") * 2 + lax.axis_index("c")

    def img_body(i, carry):
        b = wid * IMGS_PER_WORKER + i

        def in_copies(k, p):
            r0 = k * ROWS_PER_CHUNK
            cps = []
            for c in range(3):
                src = pl.multiple_of((b * 3 + c) * HW + r0 * W, 8)
                cps.append(pltpu.make_async_copy(
                    images_hbm.at[pl.ds(src, P)],
                    row_v.at[p, c, pl.ds(0, P)], SF[p]))
                nxt = jnp.minimum(r0 + ROWS_PER_CHUNK, H - 1) * W
                srcn = pl.multiple_of((b * 3 + c) * HW + nxt, 8)
                cps.append(pltpu.make_async_copy(
                    images_hbm.at[pl.ds(srcn, W)],
                    row_v.at[p, c, pl.ds(P, W)], SF[p]))
            return cps

        def tab_out_copy(k, p):
            dst = pl.multiple_of(b * HW + k * P, 8)
            return pltpu.make_async_copy(
                stage_v.at[p], tab_hbm.at[pl.ds(dst, P), :], SO[p])

        def interleave(p):
            @plsc.parallel_loop(0, G, unroll=4)
            def build_group(g):
                s = g * 16
                lane = lax.iota(jnp.int32, 16)
                rowidx = s + lane
                for c in range(3):
                    for t, off in enumerate((0, 1, W, W + 1)):
                        v = row_v[p, c, pl.ds(s + off, 16)]
                        plsc.store_scatter(
                            stage_v.at[p],
                            [rowidx, jnp.full((16,), 4 * c + t, jnp.int32)], v)

        for cp in in_copies(0, 0):
            cp.start()
        for cp in in_copies(1, 1):
            cp.start()

        def build_pair(j, carry):
            for p in (0, 1):
                k = 2 * j + p
                for cp in in_copies(k, p):
                    cp.wait()

                @pl.when(j >= 1)
                def _():
                    tab_out_copy(k - 2, p).wait()

                interleave(p)
                tab_out_copy(k, p).start()

                @pl.when(j < NCH // 2 - 1)
                def _():
                    for cp in in_copies(k + 2, p):
                        cp.start()
            return carry

        lax.fori_loop(0, NCH // 2, build_pair, 0)
        tab_out_copy(NCH - 2, 0).wait()
        tab_out_copy(NCH - 1, 1).wait()

        def flow_copies(k, p):
            off = k * P
            fy = pl.multiple_of(2 * b * HW + off, 8)
            fx = pl.multiple_of((2 * b + 1) * HW + off, 8)
            return [
                pltpu.make_async_copy(flows_hbm.at[pl.ds(fy, P)],
                                      fl_v.at[p, 0], SF[p]),
                pltpu.make_async_copy(flows_hbm.at[pl.ds(fx, P)],
                                      fl_v.at[p, 1], SF[p]),
            ]

        def gather_copy(p):
            return pltpu.make_async_copy(
                tab_hbm.at[idx_v.at[p]], stage_v.at[p], SG[p])

        def out_copies(k, p):
            off = k * P
            cps = []
            for c in range(3):
                dst = pl.multiple_of((3 * b + c) * HW + off, 8)
                cps.append(pltpu.make_async_copy(
                    o_v.at[p, c], out_hbm.at[pl.ds(dst, P)], SO[p]))
            return cps

        def compute(k, p):
            off = k * P

            @plsc.parallel_loop(0, G, unroll=4)
            def compute_group(g):
                s = g * 16
                lane = lax.iota(jnp.int32, 16)
                hw = off + s + lane
                yi = hw >> 9
                xi = hw & 511
                fy = fl_v[p, 0, pl.ds(s, 16)]
                fx = fl_v[p, 1, pl.ds(s, 16)]
                sy = jnp.clip(yi.astype(jnp.float32) + fy, 0.0, float(H) - 1.0)
                sx = jnp.clip(xi.astype(jnp.float32) + fx, 0.0, float(W) - 1.0)
                y0 = jnp.minimum(sy.astype(jnp.int32), H - 2)
                x0 = jnp.minimum(sx.astype(jnp.int32), W - 2)
                ay = sy - y0.astype(jnp.float32)
                ax = sx - x0.astype(jnp.float32)
                by = 1.0 - ay
                bx = 1.0 - ax
                w_v[p, 0, pl.ds(s, 16)] = bx * by
                w_v[p, 1, pl.ds(s, 16)] = ax * by
                w_v[p, 2, pl.ds(s, 16)] = bx * ay
                w_v[p, 3, pl.ds(s, 16)] = ax * ay
                idx_v[p, pl.ds(s, 16)] = b * HW + y0 * W + x0

        def combine(p):
            @plsc.parallel_loop(0, G, unroll=4)
            def combine_group(g):
                s = g * 16
                lane = lax.iota(jnp.int32, 16)
                rowidx = s + lane
                wa = w_v[p, 0, pl.ds(s, 16)]
                wc = w_v[p, 1, pl.ds(s, 16)]
                wb = w_v[p, 2, pl.ds(s, 16)]
                wd = w_v[p, 3, pl.ds(s, 16)]
                for c in range(3):
                    def tap(t):
                        col = jnp.full((16,), 4 * c + t, jnp.int32)
                        return plsc.load_gather(stage_v.at[p], [rowidx, col])
                    acc = wa * tap(0)
                    acc = acc + wc * tap(1)
                    acc = acc + wb * tap(2)
                    acc = acc + wd * tap(3)
                    o_v[p, c, pl.ds(s, 16)] = acc

        for cp in flow_copies(0, 0):
            cp.start()

        def sample_pair(j, carry):
            for p in (0, 1):
                k = 2 * j + p
                for cp in flow_copies(k, p):
                    cp.wait()
                compute(k, p)
                gather_copy(p).start()

                if p == 0:
                    for cp in flow_copies(k + 1, 1):
                        cp.start()
                else:
                    @pl.when(j < NCH // 2 - 1)
                    def _():
                        for cp in flow_copies(k + 1, 0):
                            cp.start()

                q = 1 - p
                if p == 0:
                    @pl.when(j >= 1)
                    def _():
                        gather_copy(q).wait()

                        @pl.when(j >= 2)
                        def _():
                            for cp in out_copies(k - 3, q):
                                cp.wait()

                        combine(q)
                        for cp in out_copies(k - 1, q):
                            cp.start()
                else:
                    gather_copy(q).wait()

                    @pl.when(j >= 1)
                    def _():
                        for cp in out_copies(k - 3, q):
                            cp.wait()

                    combine(q)
                    for cp in out_copies(k - 1, q):
                        cp.start()
            return carry

        lax.fori_loop(0, NCH // 2, sample_pair, 0)

        gather_copy(1).wait()
        for cp in out_copies(NCH - 3, 1):
            cp.wait()
        combine(1)
        for cp in out_copies(NCH - 1, 1):
            cp.start()
        for cp in out_copies(NCH - 2, 0):
            cp.wait()
        for cp in out_copies(NCH - 1, 1):
            cp.wait()
        return carry

    lax.fori_loop(0, IMGS_PER_WORKER, img_body, 0)


@jax.jit
def kernel(images, flows):
    images_flat = images.reshape(B * C * HW)
    flows_flat = flows.reshape(B * 2 * HW)
    mesh = plsc.VectorSubcoreMesh(core_axis_name="c", subcore_axis_name="s")
    run = functools.partial(
        pl.kernel,
        mesh=mesh,
        out_type=(
            jax.ShapeDtypeStruct((B * C * HW,), jnp.float32),
            jax.ShapeDtypeStruct((B * HW, D), jnp.float32),
        ),
        scratch_types=[
            pltpu.VMEM((2, 3, RB), jnp.float32),
            pltpu.VMEM((2, P, D), jnp.float32),
            pltpu.VMEM((2, 2, P), jnp.float32),
            pltpu.VMEM((2, 4, P), jnp.float32),
            pltpu.VMEM((2, P), jnp.int32),
            pltpu.VMEM((2, 3, P), jnp.float32),
            pltpu.SemaphoreType.DMA,
            pltpu.SemaphoreType.DMA,
            pltpu.SemaphoreType.DMA,
            pltpu.SemaphoreType.DMA,
            pltpu.SemaphoreType.DMA,
            pltpu.SemaphoreType.DMA,
        ],
        compiler_params=pltpu.CompilerParams(
            use_tc_tiling_on_sc=False, needs_layout_passes=False),
    )(_sc_warp)
    out, _tab = run(images_flat, flows_flat)
    return out.reshape(B, C, H, W)
